# baseline (speedup 1.0000x reference)
import jax
import jax.numpy as jnp
from jax import lax
from jax.experimental import pallas as pl
from jax.experimental.pallas import tpu as pltpu
from jax.experimental.pallas import tpu_sc as plsc

N = 10000
E = 160000
G = 3
NT = G * N
BE = G * E
HID = 256
NG = 64
NW = 32

_MESH = plsc.VectorSubcoreMesh(core_axis_name="c", subcore_axis_name="s")

F32 = jnp.float32


def _leaky(v):
    return jnp.maximum(v, 0.01 * v)


def _elu(v):
    return jnp.where(v > 0, v, jnp.exp(jnp.minimum(v, 0.0)) - 1.0)


def _sc_gather(table, idx, W):
    B = idx.shape[0]
    per_w = B // NW
    K = 200 if W > 64 else 3000
    nch = per_w // K
    c0f = nch % 2
    npairs = (nch - c0f) // 2

    def body(table_h, idx_h, out_h, idx_v0, idx_v1, rows_v0, rows_v1,
             sem0, sem1):
        wid = lax.axis_index("s") * 2 + lax.axis_index("c")
        base = wid * per_w

        def start(c, iv, rv, sm):
            pltpu.sync_copy(idx_h.at[pl.ds(base + c * K, K)], iv)
            pltpu.async_copy(table_h.at[iv], rv, sm)

        def drain(c, iv, rv, sm):
            pltpu.make_async_copy(table_h.at[iv], rv, sm).wait()
            pltpu.sync_copy(rv, out_h.at[pl.ds(base + c * K, K)])

        if c0f:
            start(0, idx_v0, rows_v0, sem0)
            drain(0, idx_v0, rows_v0, sem0)
        start(c0f, idx_v0, rows_v0, sem0)

        def pair(p, c):
            c0 = c0f + 2 * p
            start(c0 + 1, idx_v1, rows_v1, sem1)
            drain(c0, idx_v0, rows_v0, sem0)
            start(c0 + 2, idx_v0, rows_v0, sem0)
            drain(c0 + 1, idx_v1, rows_v1, sem1)
            return c

        lax.fori_loop(0, npairs - 1, pair, 0)
        cl = c0f + 2 * (npairs - 1)
        start(cl + 1, idx_v1, rows_v1, sem1)
        drain(cl, idx_v0, rows_v0, sem0)
        drain(cl + 1, idx_v1, rows_v1, sem1)

    return pl.kernel(
        body,
        out_type=jax.ShapeDtypeStruct((B, W), F32),
        mesh=_MESH,
        compiler_params=pltpu.CompilerParams(use_tc_tiling_on_sc=False),
        scratch_types=[
            pltpu.VMEM((K,), jnp.int32),
            pltpu.VMEM((K,), jnp.int32),
            pltpu.VMEM((K, W), F32),
            pltpu.VMEM((K, W), F32),
            pltpu.SemaphoreType.DMA,
            pltpu.SemaphoreType.DMA,
        ],
    )(table, idx)


def _sc_scatter_add(upd, idx, T, W):
    B = idx.shape[0]
    per_w = B // NW
    K = 600
    nch = per_w // K
    c0f = nch % 2
    npairs = (nch - c0f) // 2
    RT = T // 16
    CZ = 375
    nz = RT // CZ

    def body(upd_h, idx_h, out_h, idx_v0, idx_v1, upd_v0, upd_v1, buf_v,
             si0, su0, si1, su1, acc_sh):
        cid = lax.axis_index("c")
        sid = lax.axis_index("s")
        wid = sid * 2 + cid

        def zrow(r, c):
            for j in range(W // 16):
                buf_v[r, pl.ds(j * 16, 16)] = jnp.zeros((16,), F32)
            return c

        lax.fori_loop(0, CZ, zrow, 0)

        def zch(z, c):
            pltpu.sync_copy(buf_v, acc_sh.at[pl.ds(sid * RT + z * CZ, CZ)])
            return c

        lax.fori_loop(0, nz, zch, 0)
        plsc.subcore_barrier()

        base = wid * per_w

        def start(ci, iv, uv, smi, smu):
            off = base + ci * K
            pltpu.async_copy(idx_h.at[pl.ds(off, K)], iv, smi)
            pltpu.async_copy(upd_h.at[pl.ds(off, K)], uv, smu)

        def drain(ci, iv, uv, smi, smu):
            off = base + ci * K
            pltpu.make_async_copy(idx_h.at[pl.ds(off, K)], iv, smi).wait()
            pltpu.make_async_copy(upd_h.at[pl.ds(off, K)], uv, smu).wait()
            pltpu.sync_copy(uv, acc_sh.at[iv], add=True)

        if c0f:
            start(0, idx_v0, upd_v0, si0, su0)
            drain(0, idx_v0, upd_v0, si0, su0)
        start(c0f, idx_v0, upd_v0, si0, su0)

        def pair(p, c):
            c0 = c0f + 2 * p
            start(c0 + 1, idx_v1, upd_v1, si1, su1)
            drain(c0, idx_v0, upd_v0, si0, su0)
            start(c0 + 2, idx_v0, upd_v0, si0, su0)
            drain(c0 + 1, idx_v1, upd_v1, si1, su1)
            return c

        lax.fori_loop(0, npairs - 1, pair, 0)
        cl = c0f + 2 * (npairs - 1)
        start(cl + 1, idx_v1, upd_v1, si1, su1)
        drain(cl, idx_v0, upd_v0, si0, su0)
        drain(cl + 1, idx_v1, upd_v1, si1, su1)
        plsc.subcore_barrier()

        def rch(z, c):
            r0 = sid * RT + z * CZ
            pltpu.sync_copy(acc_sh.at[pl.ds(r0, CZ)], buf_v)
            pltpu.sync_copy(buf_v, out_h.at[cid].at[pl.ds(r0, CZ)])
            return c

        lax.fori_loop(0, nz, rch, 0)

    return pl.kernel(
        body,
        out_type=jax.ShapeDtypeStruct((2, T, W), F32),
        mesh=_MESH,
        compiler_params=pltpu.CompilerParams(use_tc_tiling_on_sc=False),
        scratch_types=[
            pltpu.VMEM((K,), jnp.int32),
            pltpu.VMEM((K,), jnp.int32),
            pltpu.VMEM((K, W), F32),
            pltpu.VMEM((K, W), F32),
            pltpu.VMEM((CZ, W), F32),
            pltpu.SemaphoreType.DMA,
            pltpu.SemaphoreType.DMA,
            pltpu.SemaphoreType.DMA,
            pltpu.SemaphoreType.DMA,
            pltpu.VMEM_SHARED((T, W), F32),
        ],
    )(upd, idx)


def _sc_scatter_add_multi(upds, idx, T):
    NS_ = len(upds)
    W = 32
    B = idx.shape[0]
    per_w = B // NW
    K = 600
    nch = per_w // K
    c0f = nch % 2
    npairs = (nch - c0f) // 2
    RT = T // 16
    CZ = 375
    nz = RT // CZ

    def body(*refs):
        upd_hs = refs[:NS_]
        idx_h = refs[NS_]
        out_hs = refs[NS_ + 1:2 * NS_ + 1]
        (idx_v0, idx_v1, upd_v0, upd_v1, buf_v, si0, su0, si1, su1,
         acc_sh) = refs[2 * NS_ + 1:]
        cid = lax.axis_index("c")
        sid = lax.axis_index("s")
        wid = sid * 2 + cid
        base = wid * per_w

        def zrow(r, c):
            for j in range(W // 16):
                buf_v[r, pl.ds(j * 16, 16)] = jnp.zeros((16,), F32)
            return c

        for s in range(NS_):
            upd_h = upd_hs[s]
            lax.fori_loop(0, CZ, zrow, 0)

            def zch(z, c):
                pltpu.sync_copy(buf_v,
                                acc_sh.at[pl.ds(sid * RT + z * CZ, CZ)])
                return c

            lax.fori_loop(0, nz, zch, 0)
            plsc.subcore_barrier()

            def start(ci, iv, uv, smi, smu):
                off = base + ci * K
                pltpu.async_copy(idx_h.at[pl.ds(off, K)], iv, smi)
                pltpu.async_copy(upd_h.at[pl.ds(off, K)], uv, smu)

            def drain(ci, iv, uv, smi, smu):
                off = base + ci * K
                pltpu.make_async_copy(idx_h.at[pl.ds(off, K)], iv,
                                      smi).wait()
                pltpu.make_async_copy(upd_h.at[pl.ds(off, K)], uv,
                                      smu).wait()
                pltpu.sync_copy(uv, acc_sh.at[iv], add=True)

            if c0f:
                start(0, idx_v0, upd_v0, si0, su0)
                drain(0, idx_v0, upd_v0, si0, su0)
            start(c0f, idx_v0, upd_v0, si0, su0)

            def pair(p, c):
                c0 = c0f + 2 * p
                start(c0 + 1, idx_v1, upd_v1, si1, su1)
                drain(c0, idx_v0, upd_v0, si0, su0)
                start(c0 + 2, idx_v0, upd_v0, si0, su0)
                drain(c0 + 1, idx_v1, upd_v1, si1, su1)
                return c

            lax.fori_loop(0, npairs - 1, pair, 0)
            cl = c0f + 2 * (npairs - 1)
            start(cl + 1, idx_v1, upd_v1, si1, su1)
            drain(cl, idx_v0, upd_v0, si0, su0)
            drain(cl + 1, idx_v1, upd_v1, si1, su1)
            plsc.subcore_barrier()

            def rch(z, c):
                r0 = sid * RT + z * CZ
                pltpu.sync_copy(acc_sh.at[pl.ds(r0, CZ)], buf_v)
                pltpu.sync_copy(buf_v, out_hs[s].at[cid].at[pl.ds(r0, CZ)])
                return c

            lax.fori_loop(0, nz, rch, 0)
            plsc.subcore_barrier()

    return pl.kernel(
        body,
        out_type=[jax.ShapeDtypeStruct((2, T, W), F32)] * NS_,
        mesh=_MESH,
        compiler_params=pltpu.CompilerParams(use_tc_tiling_on_sc=False),
        scratch_types=[
            pltpu.VMEM((K,), jnp.int32),
            pltpu.VMEM((K,), jnp.int32),
            pltpu.VMEM((K, W), F32),
            pltpu.VMEM((K, W), F32),
            pltpu.VMEM((CZ, W), F32),
            pltpu.SemaphoreType.DMA,
            pltpu.SemaphoreType.DMA,
            pltpu.SemaphoreType.DMA,
            pltpu.SemaphoreType.DMA,
            pltpu.VMEM_SHARED((T, W), F32),
        ],
    )(*upds, idx)


_BLK = 400


def _dot(a, b):
    return jnp.dot(a, b, preferred_element_type=F32)


def _t1(x, W1, b1, Wx, attr, W2g):
    def body(x_r, W1_r, b1_r, Wx_r, at_r, W2_r, x1_o, xg_o, y2_o, sd_o):
        x1 = _leaky(_dot(x_r[0], W1_r[0].T) + b1_r[0])
        x1_o[0] = x1
        xg_o[0] = _dot(x1, Wx_r[0].T)
        y2_o[0] = _dot(x1, W2_r[0].T)
        sd = jnp.sum(x1 * at_r[0], axis=1, keepdims=True)
        sd_o[0] = jnp.broadcast_to(sd, (_BLK, 16))

    grid = (G, N // _BLK)
    f = pl.pallas_call(
        body,
        grid=grid,
        in_specs=[
            pl.BlockSpec((1, _BLK, 128), lambda g, i: (g, i, 0)),
            pl.BlockSpec((1, HID, 128), lambda g, i: (g, 0, 0)),
            pl.BlockSpec((1, 1, HID), lambda g, i: (g, 0, 0)),
            pl.BlockSpec((1, HID, HID), lambda g, i: (g, 0, 0)),
            pl.BlockSpec((1, 1, HID), lambda g, i: (g, 0, 0)),
            pl.BlockSpec((1, HID, HID), lambda g, i: (g, 0, 0)),
        ],
        out_specs=[
            pl.BlockSpec((1, _BLK, HID), lambda g, i: (g, i, 0)),
            pl.BlockSpec((1, _BLK, HID), lambda g, i: (g, i, 0)),
            pl.BlockSpec((1, _BLK, HID), lambda g, i: (g, i, 0)),
            pl.BlockSpec((1, _BLK, 16), lambda g, i: (g, i, 0)),
        ],
        out_shape=[
            jax.ShapeDtypeStruct((G, N, HID), F32),
            jax.ShapeDtypeStruct((G, N, HID), F32),
            jax.ShapeDtypeStruct((G, N, HID), F32),
            jax.ShapeDtypeStruct((G, N, 16), F32),
        ],
    )
    return f(x, W1, b1, Wx, attr, W2g)


def _t1b(ea, We):
    EB = 1000

    def body(e_r, W_r, o_r):
        o_r[0] = _dot(e_r[0], W_r[0].T)

    f = pl.pallas_call(
        body,
        grid=(G, E // EB),
        in_specs=[
            pl.BlockSpec((1, EB, 16), lambda g, i: (g, i, 0)),
            pl.BlockSpec((1, HID, 16), lambda g, i: (g, 0, 0)),
        ],
        out_specs=pl.BlockSpec((1, EB, HID), lambda g, i: (g, i, 0)),
        out_shape=jax.ShapeDtypeStruct((G, E, HID), F32),
    )
    return f(ea, We)


def _t2(Gx, ew, sd_e, attl):
    EB = 1000
    ng = E // EB

    def body(gx_r, ew_r, sd_r, al_r, o_r):
        pre = _leaky(gx_r[...] + ew_r[...])
        z = jnp.sum(pre * al_r[0], axis=1, keepdims=True)
        a = _leaky(z + sd_r[:, 0:1])
        o_r[...] = jnp.broadcast_to(jnp.exp(a), (EB, 16))

    f = pl.pallas_call(
        body,
        grid=(G, ng),
        in_specs=[
            pl.BlockSpec((EB, HID), lambda g, i: (g * ng + i, 0)),
            pl.BlockSpec((EB, HID), lambda g, i: (g * ng + i, 0)),
            pl.BlockSpec((EB, 16), lambda g, i: (g * ng + i, 0)),
            pl.BlockSpec((1, 1, HID), lambda g, i: (g, 0, 0)),
        ],
        out_specs=pl.BlockSpec((EB, 16), lambda g, i: (g * ng + i, 0)),
        out_shape=jax.ShapeDtypeStruct((BE, 16), F32),
    )
    return f(Gx, ew, sd_e, attl)


def _t4(Gy, ex16):
    EB = 1000
    NS = 8

    def body(gy_r, ex_r, *qs):
        row = gy_r[...] * ex_r[:, 0:1]
        for s in range(NS):
            qs[s][...] = row[:, 32 * s:32 * s + 32]

    f = pl.pallas_call(
        body,
        grid=(BE // EB,),
        in_specs=[
            pl.BlockSpec((EB, HID), lambda i: (i, 0)),
            pl.BlockSpec((EB, 16), lambda i: (i, 0)),
        ],
        out_specs=[pl.BlockSpec((EB, 32), lambda i: (i, 0))] * NS,
        out_shape=[jax.ShapeDtypeStruct((BE, 32), F32)] * NS,
    )
    return f(Gy, ex16)


def _gru_body(Wih, Whh, bih, bhh, xin, h):
    gi = _dot(xin, Wih.T) + bih
    gh = _dot(h, Whh.T) + bhh
    i_r, i_z, i_n = jnp.split(gi, 3, axis=1)
    h_r, h_z, h_n = jnp.split(gh, 3, axis=1)
    r = jax.nn.sigmoid(i_r + h_r)
    z = jax.nn.sigmoid(i_z + h_z)
    n = jnp.tanh(i_n + r * h_n)
    return (1.0 - z) * n + z * h


def _t5(hq, denp, bias, xprev, Wih, Whh, bih, bhh, Wn, asv, adv):
    def body(h0, h1, h2, h3, h4, h5, h6, h7, dn_r, b_r, xp_r, Wih_r, Whh_r,
             bih_r, bhh_r, Wn_r, as_r, ad_r, xn_o, xs_o, s_o, d_o):
        hs = [h0, h1, h2, h3, h4, h5, h6, h7]
        den = dn_r[0, :, 0:1] + dn_r[1, :, 0:1] + 1e-16
        h = (jnp.concatenate([q[0] + q[1] for q in hs], axis=1) / den
             + b_r[0])
        xn = jax.nn.relu(
            _gru_body(Wih_r[0], Whh_r[0], bih_r[0], bhh_r[0], _elu(h), xp_r[0]))
        xn_o[0] = xn
        xs = _dot(xn, Wn_r[0].T)
        xs_o[0] = xs
        s_o[0] = jnp.broadcast_to(
            jnp.sum(xs * as_r[0], axis=1, keepdims=True), (_BLK, 16))
        d_o[0] = jnp.broadcast_to(
            jnp.sum(xs * ad_r[0], axis=1, keepdims=True), (_BLK, 16))

    nb = N // _BLK
    hspec = pl.BlockSpec((2, _BLK, 32), lambda g, i: (0, g * nb + i, 0))
    wvec = pl.BlockSpec((1, 1, HID), lambda g, i: (g, 0, 0))
    f = pl.pallas_call(
        body,
        grid=(G, nb),
        in_specs=[
            hspec, hspec, hspec, hspec, hspec, hspec, hspec, hspec,
            pl.BlockSpec((2, _BLK, 16), lambda g, i: (0, g * nb + i, 0)),
            wvec,
            pl.BlockSpec((1, _BLK, HID), lambda g, i: (g, i, 0)),
            pl.BlockSpec((1, 3 * HID, HID), lambda g, i: (g, 0, 0)),
            pl.BlockSpec((1, 3 * HID, HID), lambda g, i: (g, 0, 0)),
            pl.BlockSpec((1, 1, 3 * HID), lambda g, i: (g, 0, 0)),
            pl.BlockSpec((1, 1, 3 * HID), lambda g, i: (g, 0, 0)),
            pl.BlockSpec((1, HID, HID), lambda g, i: (g, 0, 0)),
            wvec, wvec,
        ],
        out_specs=[
            pl.BlockSpec((1, _BLK, HID), lambda g, i: (g, i, 0)),
            pl.BlockSpec((1, _BLK, HID), lambda g, i: (g, i, 0)),
            pl.BlockSpec((1, _BLK, 16), lambda g, i: (g, i, 0)),
            pl.BlockSpec((1, _BLK, 16), lambda g, i: (g, i, 0)),
        ],
        out_shape=[
            jax.ShapeDtypeStruct((G, N, HID), F32),
            jax.ShapeDtypeStruct((G, N, HID), F32),
            jax.ShapeDtypeStruct((G, N, 16), F32),
            jax.ShapeDtypeStruct((G, N, 16), F32),
        ],
    )
    return f(*hq, denp, bias, xprev, Wih, Whh, bih, bhh, Wn, asv, adv)


def _t6(gs, gd):
    EB = 1000

    def body(s_r, d_r, o_r):
        o_r[...] = jnp.exp(_leaky(s_r[...] + d_r[...]))

    f = pl.pallas_call(
        body,
        grid=(BE // EB,),
        in_specs=[
            pl.BlockSpec((EB, 16), lambda i: (i, 0)),
            pl.BlockSpec((EB, 16), lambda i: (i, 0)),
        ],
        out_specs=pl.BlockSpec((EB, 16), lambda i: (i, 0)),
        out_shape=jax.ShapeDtypeStruct((BE, 16), F32),
    )
    return f(gs, gd)


def _t8b(x4, batch):
    def body(x_r, b_r, o_r):
        ind = (lax.broadcasted_iota(jnp.int32, (NG, N), 0)
               == b_r[0]).astype(F32)
        o_r[0] = jax.nn.relu(_dot(ind, x_r[0]))

    f = pl.pallas_call(
        body,
        grid=(G,),
        in_specs=[
            pl.BlockSpec((1, N, HID), lambda g: (g, 0, 0)),
            pl.BlockSpec((1, 1, N), lambda g: (g, 0, 0)),
        ],
        out_specs=pl.BlockSpec((1, NG, HID), lambda g: (g, 0, 0)),
        out_shape=jax.ShapeDtypeStruct((G, NG, HID), F32),
    )
    return f(x4, batch)


def _t9(out0, xs_mol, asn16, batch, molW, attd, molb, Wih, Whh, bih, bhh,
        l2W, l2b):
    def body(o0_r, xs_r, an_r, b_r, mW_r, ad_r, mb_r, Wih_r, Whh_r, bih_r,
             bhh_r, l2_r, l2b_r, o_r):
        bat = b_r[0]
        ind = (lax.broadcasted_iota(jnp.int32, (NG, N), 0) == bat).astype(F32)
        xs = xs_r[0]
        a_sn = an_r[0][:, 0:1]
        out = o0_r[0]
        for _ in range(2):
            xd = _dot(out, mW_r[0].T)
            ad64 = _dot(xd, ad_r[0].reshape(HID, 1))
            adn = jnp.sum(ind * ad64, axis=0).reshape(N, 1)
            ex = jnp.exp(_leaky(a_sn + adn))
            den = _dot(ind, ex)
            recn = jnp.sum(ind * (1.0 / (den + 1e-16)), axis=0).reshape(N, 1)
            h = _dot(ind, xs * (ex * recn)) + mb_r[0]
            out = jax.nn.relu(
                _gru_body(Wih_r[0], Whh_r[0], bih_r[0], bhh_r[0], _elu(h),
                          out))
        o_r[0] = _dot(out, l2_r[0].T) + l2b_r[0]

    wvec = pl.BlockSpec((1, 1, HID), lambda g: (g, 0, 0))
    f = pl.pallas_call(
        body,
        grid=(G,),
        in_specs=[
            pl.BlockSpec((1, NG, HID), lambda g: (g, 0, 0)),
            pl.BlockSpec((1, N, HID), lambda g: (g, 0, 0)),
            pl.BlockSpec((1, N, 16), lambda g: (g, 0, 0)),
            pl.BlockSpec((1, 1, N), lambda g: (g, 0, 0)),
            pl.BlockSpec((1, HID, HID), lambda g: (g, 0, 0)),
            wvec, wvec,
            pl.BlockSpec((1, 3 * HID, HID), lambda g: (g, 0, 0)),
            pl.BlockSpec((1, 3 * HID, HID), lambda g: (g, 0, 0)),
            pl.BlockSpec((1, 1, 3 * HID), lambda g: (g, 0, 0)),
            pl.BlockSpec((1, 1, 3 * HID), lambda g: (g, 0, 0)),
            pl.BlockSpec((1, 64, HID), lambda g: (g, 0, 0)),
            pl.BlockSpec((1, 1, 64), lambda g: (g, 0, 0)),
        ],
        out_specs=pl.BlockSpec((1, NG, 64), lambda g: (g, 0, 0)),
        out_shape=jax.ShapeDtypeStruct((G, NG, 64), F32),
        compiler_params=pltpu.CompilerParams(
            vmem_limit_bytes=100 * 1024 * 1024),
    )
    return f(out0, xs_mol, asn16, batch, molW, attd, molb, Wih, Whh, bih,
             bhh, l2W, l2b)


def _t10(o, fc1W, fc1b, fc2W, fc2b):
    def body(o_r, w1_r, b1_r, w2_r, b2_r, out_r):
        z = jnp.concatenate([o_r[0], o_r[1], o_r[2]], axis=1)
        z1 = jax.nn.relu(_dot(z, w1_r[...].T) + b1_r[...])
        out_r[...] = (jnp.sum(z1 * w2_r[...], axis=1, keepdims=True)
                      + b2_r[...])

    f = pl.pallas_call(
        body,
        grid=(1,),
        in_specs=[
            pl.BlockSpec((G, NG, 64), lambda i: (0, 0, 0)),
            pl.BlockSpec((32, 192), lambda i: (0, 0)),
            pl.BlockSpec((1, 32), lambda i: (0, 0)),
            pl.BlockSpec((1, 32), lambda i: (0, 0)),
            pl.BlockSpec((1, 1), lambda i: (0, 0)),
        ],
        out_specs=pl.BlockSpec((NG, 1), lambda i: (0, 0)),
        out_shape=jax.ShapeDtypeStruct((NG, 1), F32),
    )
    return f(o, fc1W, fc1b, fc2W, fc2b)


def _edge_layer(ex16, table, dsts, srcs):
    denp = _sc_scatter_add(ex16, dsts, NT, 16)
    Gy = _sc_gather(table, srcs, HID)
    slices = _t4(Gy, ex16)
    return _sc_scatter_add_multi(slices, dsts, NT), denp


def kernel(x_inter, edge_index_inter, edge_attr_inter, batch_inter,
           x_intra1, edge_index_intra1, edge_attr_intra1, batch_intra1,
           x_intra2, edge_index_intra2, edge_attr_intra2, batch_intra2,
           params):
    ps = [params['g1'], params['g2'], params['g3']]

    x = jnp.stack([x_inter, x_intra1, x_intra2])
    ea = jnp.stack([edge_attr_inter, edge_attr_intra1, edge_attr_intra2])
    batch = jnp.stack([batch_inter, batch_intra1, batch_intra2])[:, None, :]
    eis = [edge_index_inter, edge_index_intra1, edge_index_intra2]
    srcs = jnp.concatenate([eis[g][0] + g * N for g in range(G)])
    dsts = jnp.concatenate([eis[g][1] + g * N for g in range(G)])

    def st(path, vec=False):
        def get(p):
            v = p
            for k in path:
                v = v[k]
            return v
        s = jnp.stack([get(p) for p in ps])
        return s[:, None, :] if vec else s

    W1 = st(['lin1_W'])
    b1 = st(['lin1_b'], vec=True)
    gateW = st(['gate', 'lin1_W'])
    Wx = gateW[:, :, :HID]
    We = gateW[:, :, HID:]
    attl = st(['gate', 'att_l'], vec=True)
    attr = st(['gate', 'att_r'], vec=True)
    W2g = st(['gate', 'lin2_W'])
    gate_b = st(['gate', 'bias'], vec=True)

    def conv_params(i):
        if i < 2:
            c = [p['atom_convs'][i] for p in ps]
        else:
            c = [p['mol_conv'] for p in ps]
        return (jnp.stack([q['lin_W'] for q in c]),
                jnp.stack([q['att_src'] for q in c])[:, None, :],
                jnp.stack([q['att_dst'] for q in c])[:, None, :],
                jnp.stack([q['bias'] for q in c])[:, None, :])

    def gru_params(i):
        if i == 0:
            g = [p['gru0'] for p in ps]
        elif i < 3:
            g = [p['atom_grus'][i - 1] for p in ps]
        else:
            g = [p['mol_gru'] for p in ps]
        return (jnp.stack([q['W_ih'] for q in g]),
                jnp.stack([q['W_hh'] for q in g]),
                jnp.stack([q['b_ih'] for q in g])[:, None, :],
                jnp.stack([q['b_hh'] for q in g])[:, None, :])

    x1, xg, y2, sd16 = _t1(x, W1, b1, Wx, attr, W2g)
    ew = _t1b(ea, We).reshape(BE, HID)

    Gx = _sc_gather(xg.reshape(NT, HID), srcs, HID)
    sd_e = _sc_gather(sd16.reshape(NT, 16), dsts, 16)
    ex16 = _t2(Gx, ew, sd_e, attl)
    hq, denp = _edge_layer(ex16, y2.reshape(NT, HID), dsts, srcs)

    cW0, cs0, cd0, _ = conv_params(0)
    xcur, xs, as16, ad16 = _t5(hq, denp, gate_b, x1, *gru_params(0),
                               cW0, cs0, cd0)

    for layer in range(2):
        gs = _sc_gather(as16.reshape(NT, 16), srcs, 16)
        gd = _sc_gather(ad16.reshape(NT, 16), dsts, 16)
        ex16 = _t6(gs, gd)
        hq, denp = _edge_layer(ex16, xs.reshape(NT, HID), dsts, srcs)
        cWn, csn, cdn, _ = conv_params(layer + 1)
        _, _, _, conv_bias = conv_params(layer)
        xcur, xs, as16, ad16 = _t5(hq, denp, conv_bias, xcur,
                                   *gru_params(layer + 1), cWn, csn, cdn)

    molW, _, mol_ad, mol_b = conv_params(2)
    out0 = _t8b(xcur, batch)
    o = _t9(out0, xs, as16, batch, molW, mol_ad, mol_b, *gru_params(3),
            st(['lin2_W']), st(['lin2_b'], vec=True))

    return _t10(o, params['fc1_W'], params['fc1_b'].reshape(1, 32),
                params['fc2_W'], params['fc2_b'].reshape(1, 1))


# speedup vs baseline: 1.0572x; 1.0572x over previous
import jax
import jax.numpy as jnp
from jax import lax
from jax.experimental import pallas as pl
from jax.experimental.pallas import tpu as pltpu
from jax.experimental.pallas import tpu_sc as plsc

N = 10000
E = 160000
G = 3
NT = G * N
BE = G * E
HID = 256
NG = 64
NW = 32

_MESH = plsc.VectorSubcoreMesh(core_axis_name="c", subcore_axis_name="s")

F32 = jnp.float32


def _leaky(v):
    return jnp.maximum(v, 0.01 * v)


def _elu(v):
    return jnp.where(v > 0, v, jnp.exp(jnp.minimum(v, 0.0)) - 1.0)


def _sc_gather(table, idx, W):
    B = idx.shape[0]
    per_w = B // NW
    K = 200 if W > 64 else 3000
    nch = per_w // K
    c0f = nch % 2
    npairs = (nch - c0f) // 2
    dt = table.dtype

    def body(table_h, idx_h, out_h, idx_v0, idx_v1, rows_v0, rows_v1,
             sem0, sem1):
        wid = lax.axis_index("s") * 2 + lax.axis_index("c")
        base = wid * per_w

        def start(c, iv, rv, sm):
            pltpu.sync_copy(idx_h.at[pl.ds(base + c * K, K)], iv)
            pltpu.async_copy(table_h.at[iv], rv, sm)

        def drain(c, iv, rv, sm):
            pltpu.make_async_copy(table_h.at[iv], rv, sm).wait()
            pltpu.sync_copy(rv, out_h.at[pl.ds(base + c * K, K)])

        if c0f:
            start(0, idx_v0, rows_v0, sem0)
            drain(0, idx_v0, rows_v0, sem0)
        start(c0f, idx_v0, rows_v0, sem0)

        def pair(p, c):
            c0 = c0f + 2 * p
            start(c0 + 1, idx_v1, rows_v1, sem1)
            drain(c0, idx_v0, rows_v0, sem0)
            start(c0 + 2, idx_v0, rows_v0, sem0)
            drain(c0 + 1, idx_v1, rows_v1, sem1)
            return c

        lax.fori_loop(0, npairs - 1, pair, 0)
        cl = c0f + 2 * (npairs - 1)
        start(cl + 1, idx_v1, rows_v1, sem1)
        drain(cl, idx_v0, rows_v0, sem0)
        drain(cl + 1, idx_v1, rows_v1, sem1)

    return pl.kernel(
        body,
        out_type=jax.ShapeDtypeStruct((B, W), dt),
        mesh=_MESH,
        compiler_params=pltpu.CompilerParams(use_tc_tiling_on_sc=False),
        scratch_types=[
            pltpu.VMEM((K,), jnp.int32),
            pltpu.VMEM((K,), jnp.int32),
            pltpu.VMEM((K, W), dt),
            pltpu.VMEM((K, W), dt),
            pltpu.SemaphoreType.DMA,
            pltpu.SemaphoreType.DMA,
        ],
    )(table, idx)


def _sc_scatter_add(upd, idx, T, W):
    B = idx.shape[0]
    per_w = B // NW
    K = 600
    nch = per_w // K
    c0f = nch % 2
    npairs = (nch - c0f) // 2
    RT = T // 16
    CZ = 375
    nz = RT // CZ

    def body(upd_h, idx_h, out_h, idx_v0, idx_v1, upd_v0, upd_v1, buf_v,
             si0, su0, si1, su1, acc_sh):
        cid = lax.axis_index("c")
        sid = lax.axis_index("s")
        wid = sid * 2 + cid

        def zrow(r, c):
            for j in range(W // 16):
                buf_v[r, pl.ds(j * 16, 16)] = jnp.zeros((16,), F32)
            return c

        lax.fori_loop(0, CZ, zrow, 0)

        def zch(z, c):
            pltpu.sync_copy(buf_v, acc_sh.at[pl.ds(sid * RT + z * CZ, CZ)])
            return c

        lax.fori_loop(0, nz, zch, 0)
        plsc.subcore_barrier()

        base = wid * per_w

        def start(ci, iv, uv, smi, smu):
            off = base + ci * K
            pltpu.async_copy(idx_h.at[pl.ds(off, K)], iv, smi)
            pltpu.async_copy(upd_h.at[pl.ds(off, K)], uv, smu)

        def drain(ci, iv, uv, smi, smu):
            off = base + ci * K
            pltpu.make_async_copy(idx_h.at[pl.ds(off, K)], iv, smi).wait()
            pltpu.make_async_copy(upd_h.at[pl.ds(off, K)], uv, smu).wait()
            pltpu.sync_copy(uv, acc_sh.at[iv], add=True)

        if c0f:
            start(0, idx_v0, upd_v0, si0, su0)
            drain(0, idx_v0, upd_v0, si0, su0)
        start(c0f, idx_v0, upd_v0, si0, su0)

        def pair(p, c):
            c0 = c0f + 2 * p
            start(c0 + 1, idx_v1, upd_v1, si1, su1)
            drain(c0, idx_v0, upd_v0, si0, su0)
            start(c0 + 2, idx_v0, upd_v0, si0, su0)
            drain(c0 + 1, idx_v1, upd_v1, si1, su1)
            return c

        lax.fori_loop(0, npairs - 1, pair, 0)
        cl = c0f + 2 * (npairs - 1)
        start(cl + 1, idx_v1, upd_v1, si1, su1)
        drain(cl, idx_v0, upd_v0, si0, su0)
        drain(cl + 1, idx_v1, upd_v1, si1, su1)
        plsc.subcore_barrier()

        def rch(z, c):
            r0 = sid * RT + z * CZ
            pltpu.sync_copy(acc_sh.at[pl.ds(r0, CZ)], buf_v)
            pltpu.sync_copy(buf_v, out_h.at[cid].at[pl.ds(r0, CZ)])
            return c

        lax.fori_loop(0, nz, rch, 0)

    return pl.kernel(
        body,
        out_type=jax.ShapeDtypeStruct((2, T, W), F32),
        mesh=_MESH,
        compiler_params=pltpu.CompilerParams(use_tc_tiling_on_sc=False),
        scratch_types=[
            pltpu.VMEM((K,), jnp.int32),
            pltpu.VMEM((K,), jnp.int32),
            pltpu.VMEM((K, W), F32),
            pltpu.VMEM((K, W), F32),
            pltpu.VMEM((CZ, W), F32),
            pltpu.SemaphoreType.DMA,
            pltpu.SemaphoreType.DMA,
            pltpu.SemaphoreType.DMA,
            pltpu.SemaphoreType.DMA,
            pltpu.VMEM_SHARED((T, W), F32),
        ],
    )(upd, idx)


def _sc_scatter_add_multi(upds, idx, T):
    NS_ = len(upds)
    W = 32
    B = idx.shape[0]
    per_w = B // NW
    K = 600
    nch = per_w // K
    c0f = nch % 2
    npairs = (nch - c0f) // 2
    RT = T // 16
    CZ = 375
    nz = RT // CZ

    def body(*refs):
        upd_hs = refs[:NS_]
        idx_h = refs[NS_]
        out_hs = refs[NS_ + 1:2 * NS_ + 1]
        (idx_v0, idx_v1, upd_v0, upd_v1, buf_v, si0, su0, si1, su1,
         acc_sh) = refs[2 * NS_ + 1:]
        cid = lax.axis_index("c")
        sid = lax.axis_index("s")
        wid = sid * 2 + cid
        base = wid * per_w

        def zrow(r, c):
            for j in range(W // 16):
                buf_v[r, pl.ds(j * 16, 16)] = jnp.zeros((16,), F32)
            return c

        for s in range(NS_):
            upd_h = upd_hs[s]
            lax.fori_loop(0, CZ, zrow, 0)

            def zch(z, c):
                pltpu.sync_copy(buf_v,
                                acc_sh.at[pl.ds(sid * RT + z * CZ, CZ)])
                return c

            lax.fori_loop(0, nz, zch, 0)
            plsc.subcore_barrier()

            def start(ci, iv, uv, smi, smu):
                off = base + ci * K
                pltpu.async_copy(idx_h.at[pl.ds(off, K)], iv, smi)
                pltpu.async_copy(upd_h.at[pl.ds(off, K)], uv, smu)

            def drain(ci, iv, uv, smi, smu):
                off = base + ci * K
                pltpu.make_async_copy(idx_h.at[pl.ds(off, K)], iv,
                                      smi).wait()
                pltpu.make_async_copy(upd_h.at[pl.ds(off, K)], uv,
                                      smu).wait()
                pltpu.sync_copy(uv, acc_sh.at[iv], add=True)

            if c0f:
                start(0, idx_v0, upd_v0, si0, su0)
                drain(0, idx_v0, upd_v0, si0, su0)
            start(c0f, idx_v0, upd_v0, si0, su0)

            def pair(p, c):
                c0 = c0f + 2 * p
                start(c0 + 1, idx_v1, upd_v1, si1, su1)
                drain(c0, idx_v0, upd_v0, si0, su0)
                start(c0 + 2, idx_v0, upd_v0, si0, su0)
                drain(c0 + 1, idx_v1, upd_v1, si1, su1)
                return c

            lax.fori_loop(0, npairs - 1, pair, 0)
            cl = c0f + 2 * (npairs - 1)
            start(cl + 1, idx_v1, upd_v1, si1, su1)
            drain(cl, idx_v0, upd_v0, si0, su0)
            drain(cl + 1, idx_v1, upd_v1, si1, su1)
            plsc.subcore_barrier()

            def rch(z, c):
                r0 = sid * RT + z * CZ
                pltpu.sync_copy(acc_sh.at[pl.ds(r0, CZ)], buf_v)
                pltpu.sync_copy(buf_v, out_hs[s].at[cid].at[pl.ds(r0, CZ)])
                return c

            lax.fori_loop(0, nz, rch, 0)
            plsc.subcore_barrier()

    return pl.kernel(
        body,
        out_type=[jax.ShapeDtypeStruct((2, T, W), F32)] * NS_,
        mesh=_MESH,
        compiler_params=pltpu.CompilerParams(use_tc_tiling_on_sc=False),
        scratch_types=[
            pltpu.VMEM((K,), jnp.int32),
            pltpu.VMEM((K,), jnp.int32),
            pltpu.VMEM((K, W), F32),
            pltpu.VMEM((K, W), F32),
            pltpu.VMEM((CZ, W), F32),
            pltpu.SemaphoreType.DMA,
            pltpu.SemaphoreType.DMA,
            pltpu.SemaphoreType.DMA,
            pltpu.SemaphoreType.DMA,
            pltpu.VMEM_SHARED((T, W), F32),
        ],
    )(*upds, idx)


_BLK = 400


def _dot(a, b):
    return jnp.dot(a, b, preferred_element_type=F32)


def _t1(x, W1, b1, Wx, attr, W2g):
    def body(x_r, W1_r, b1_r, Wx_r, at_r, W2_r, x1_o, xg_o, y2_o, sd_o):
        x1 = _leaky(_dot(x_r[0], W1_r[0].T) + b1_r[0])
        x1_o[0] = x1
        xg_o[0] = _dot(x1, Wx_r[0].T).astype(jnp.bfloat16)
        y2_o[0] = _dot(x1, W2_r[0].T).astype(jnp.bfloat16)
        sd = jnp.sum(x1 * at_r[0], axis=1, keepdims=True)
        sd_o[0] = jnp.broadcast_to(sd, (_BLK, 16))

    grid = (G, N // _BLK)
    f = pl.pallas_call(
        body,
        grid=grid,
        in_specs=[
            pl.BlockSpec((1, _BLK, 128), lambda g, i: (g, i, 0)),
            pl.BlockSpec((1, HID, 128), lambda g, i: (g, 0, 0)),
            pl.BlockSpec((1, 1, HID), lambda g, i: (g, 0, 0)),
            pl.BlockSpec((1, HID, HID), lambda g, i: (g, 0, 0)),
            pl.BlockSpec((1, 1, HID), lambda g, i: (g, 0, 0)),
            pl.BlockSpec((1, HID, HID), lambda g, i: (g, 0, 0)),
        ],
        out_specs=[
            pl.BlockSpec((1, _BLK, HID), lambda g, i: (g, i, 0)),
            pl.BlockSpec((1, _BLK, HID), lambda g, i: (g, i, 0)),
            pl.BlockSpec((1, _BLK, HID), lambda g, i: (g, i, 0)),
            pl.BlockSpec((1, _BLK, 16), lambda g, i: (g, i, 0)),
        ],
        out_shape=[
            jax.ShapeDtypeStruct((G, N, HID), F32),
            jax.ShapeDtypeStruct((G, N, HID), jnp.bfloat16),
            jax.ShapeDtypeStruct((G, N, HID), jnp.bfloat16),
            jax.ShapeDtypeStruct((G, N, 16), F32),
        ],
    )
    return f(x, W1, b1, Wx, attr, W2g)


def _t1b(ea, We):
    EB = 1000

    def body(e_r, W_r, o_r):
        o_r[0] = _dot(e_r[0], W_r[0].T)

    f = pl.pallas_call(
        body,
        grid=(G, E // EB),
        in_specs=[
            pl.BlockSpec((1, EB, 16), lambda g, i: (g, i, 0)),
            pl.BlockSpec((1, HID, 16), lambda g, i: (g, 0, 0)),
        ],
        out_specs=pl.BlockSpec((1, EB, HID), lambda g, i: (g, i, 0)),
        out_shape=jax.ShapeDtypeStruct((G, E, HID), F32),
    )
    return f(ea, We)


def _t2(Gx, ew, sd_e, attl):
    EB = 1000
    ng = E // EB

    def body(gx_r, ew_r, sd_r, al_r, o_r):
        pre = _leaky(gx_r[...].astype(F32) + ew_r[...])
        z = jnp.sum(pre * al_r[0], axis=1, keepdims=True)
        a = _leaky(z + sd_r[:, 0:1])
        o_r[...] = jnp.broadcast_to(jnp.exp(a), (EB, 16))

    f = pl.pallas_call(
        body,
        grid=(G, ng),
        in_specs=[
            pl.BlockSpec((EB, HID), lambda g, i: (g * ng + i, 0)),
            pl.BlockSpec((EB, HID), lambda g, i: (g * ng + i, 0)),
            pl.BlockSpec((EB, 16), lambda g, i: (g * ng + i, 0)),
            pl.BlockSpec((1, 1, HID), lambda g, i: (g, 0, 0)),
        ],
        out_specs=pl.BlockSpec((EB, 16), lambda g, i: (g * ng + i, 0)),
        out_shape=jax.ShapeDtypeStruct((BE, 16), F32),
    )
    return f(Gx, ew, sd_e, attl)


def _t4(Gy, ex16):
    EB = 1000
    NS = 8

    def body(gy_r, ex_r, *qs):
        row = gy_r[...].astype(F32) * ex_r[:, 0:1]
        for s in range(NS):
            qs[s][...] = row[:, 32 * s:32 * s + 32]

    f = pl.pallas_call(
        body,
        grid=(BE // EB,),
        in_specs=[
            pl.BlockSpec((EB, HID), lambda i: (i, 0)),
            pl.BlockSpec((EB, 16), lambda i: (i, 0)),
        ],
        out_specs=[pl.BlockSpec((EB, 32), lambda i: (i, 0))] * NS,
        out_shape=[jax.ShapeDtypeStruct((BE, 32), F32)] * NS,
    )
    return f(Gy, ex16)


def _gru_body(Wih, Whh, bih, bhh, xin, h):
    gi = _dot(xin, Wih.T) + bih
    gh = _dot(h, Whh.T) + bhh
    i_r, i_z, i_n = jnp.split(gi, 3, axis=1)
    h_r, h_z, h_n = jnp.split(gh, 3, axis=1)
    r = jax.nn.sigmoid(i_r + h_r)
    z = jax.nn.sigmoid(i_z + h_z)
    n = jnp.tanh(i_n + r * h_n)
    return (1.0 - z) * n + z * h


def _t5(hq, denp, bias, xprev, Wih, Whh, bih, bhh, Wn, asv, adv):
    def body(h0, h1, h2, h3, h4, h5, h6, h7, dn_r, b_r, xp_r, Wih_r, Whh_r,
             bih_r, bhh_r, Wn_r, as_r, ad_r, xn_o, xs_o, s_o, d_o):
        hs = [h0, h1, h2, h3, h4, h5, h6, h7]
        den = dn_r[0, :, 0:1] + dn_r[1, :, 0:1] + 1e-16
        h = (jnp.concatenate([q[0] + q[1] for q in hs], axis=1) / den
             + b_r[0])
        xn = jax.nn.relu(
            _gru_body(Wih_r[0], Whh_r[0], bih_r[0], bhh_r[0], _elu(h), xp_r[0]))
        xn_o[0] = xn
        xs = _dot(xn, Wn_r[0].T)
        xs_o[0] = xs.astype(jnp.bfloat16)
        s_o[0] = jnp.broadcast_to(
            jnp.sum(xs * as_r[0], axis=1, keepdims=True), (_BLK, 16))
        d_o[0] = jnp.broadcast_to(
            jnp.sum(xs * ad_r[0], axis=1, keepdims=True), (_BLK, 16))

    nb = N // _BLK
    hspec = pl.BlockSpec((2, _BLK, 32), lambda g, i: (0, g * nb + i, 0))
    wvec = pl.BlockSpec((1, 1, HID), lambda g, i: (g, 0, 0))
    f = pl.pallas_call(
        body,
        grid=(G, nb),
        in_specs=[
            hspec, hspec, hspec, hspec, hspec, hspec, hspec, hspec,
            pl.BlockSpec((2, _BLK, 16), lambda g, i: (0, g * nb + i, 0)),
            wvec,
            pl.BlockSpec((1, _BLK, HID), lambda g, i: (g, i, 0)),
            pl.BlockSpec((1, 3 * HID, HID), lambda g, i: (g, 0, 0)),
            pl.BlockSpec((1, 3 * HID, HID), lambda g, i: (g, 0, 0)),
            pl.BlockSpec((1, 1, 3 * HID), lambda g, i: (g, 0, 0)),
            pl.BlockSpec((1, 1, 3 * HID), lambda g, i: (g, 0, 0)),
            pl.BlockSpec((1, HID, HID), lambda g, i: (g, 0, 0)),
            wvec, wvec,
        ],
        out_specs=[
            pl.BlockSpec((1, _BLK, HID), lambda g, i: (g, i, 0)),
            pl.BlockSpec((1, _BLK, HID), lambda g, i: (g, i, 0)),
            pl.BlockSpec((1, _BLK, 16), lambda g, i: (g, i, 0)),
            pl.BlockSpec((1, _BLK, 16), lambda g, i: (g, i, 0)),
        ],
        out_shape=[
            jax.ShapeDtypeStruct((G, N, HID), F32),
            jax.ShapeDtypeStruct((G, N, HID), jnp.bfloat16),
            jax.ShapeDtypeStruct((G, N, 16), F32),
            jax.ShapeDtypeStruct((G, N, 16), F32),
        ],
    )
    return f(*hq, denp, bias, xprev, Wih, Whh, bih, bhh, Wn, asv, adv)


def _t6(gs, gd):
    EB = 1000

    def body(s_r, d_r, o_r):
        o_r[...] = jnp.exp(_leaky(s_r[...] + d_r[...]))

    f = pl.pallas_call(
        body,
        grid=(BE // EB,),
        in_specs=[
            pl.BlockSpec((EB, 16), lambda i: (i, 0)),
            pl.BlockSpec((EB, 16), lambda i: (i, 0)),
        ],
        out_specs=pl.BlockSpec((EB, 16), lambda i: (i, 0)),
        out_shape=jax.ShapeDtypeStruct((BE, 16), F32),
    )
    return f(gs, gd)


def _t8b(x4, batch):
    def body(x_r, b_r, o_r):
        ind = (lax.broadcasted_iota(jnp.int32, (NG, N), 0)
               == b_r[0]).astype(F32)
        o_r[0] = jax.nn.relu(_dot(ind, x_r[0]))

    f = pl.pallas_call(
        body,
        grid=(G,),
        in_specs=[
            pl.BlockSpec((1, N, HID), lambda g: (g, 0, 0)),
            pl.BlockSpec((1, 1, N), lambda g: (g, 0, 0)),
        ],
        out_specs=pl.BlockSpec((1, NG, HID), lambda g: (g, 0, 0)),
        out_shape=jax.ShapeDtypeStruct((G, NG, HID), F32),
    )
    return f(x4, batch)


def _t9(out0, xs_mol, asn16, batch, molW, attd, molb, Wih, Whh, bih, bhh,
        l2W, l2b):
    def body(o0_r, xs_r, an_r, b_r, mW_r, ad_r, mb_r, Wih_r, Whh_r, bih_r,
             bhh_r, l2_r, l2b_r, o_r):
        bat = b_r[0]
        ind = (lax.broadcasted_iota(jnp.int32, (NG, N), 0) == bat).astype(F32)
        xs = xs_r[0].astype(F32)
        a_sn = an_r[0][:, 0:1]
        out = o0_r[0]
        for _ in range(2):
            xd = _dot(out, mW_r[0].T)
            ad64 = _dot(xd, ad_r[0].reshape(HID, 1))
            adn = jnp.sum(ind * ad64, axis=0).reshape(N, 1)
            ex = jnp.exp(_leaky(a_sn + adn))
            den = _dot(ind, ex)
            recn = jnp.sum(ind * (1.0 / (den + 1e-16)), axis=0).reshape(N, 1)
            h = _dot(ind, xs * (ex * recn)) + mb_r[0]
            out = jax.nn.relu(
                _gru_body(Wih_r[0], Whh_r[0], bih_r[0], bhh_r[0], _elu(h),
                          out))
        o_r[0] = _dot(out, l2_r[0].T) + l2b_r[0]

    wvec = pl.BlockSpec((1, 1, HID), lambda g: (g, 0, 0))
    f = pl.pallas_call(
        body,
        grid=(G,),
        in_specs=[
            pl.BlockSpec((1, NG, HID), lambda g: (g, 0, 0)),
            pl.BlockSpec((1, N, HID), lambda g: (g, 0, 0)),
            pl.BlockSpec((1, N, 16), lambda g: (g, 0, 0)),
            pl.BlockSpec((1, 1, N), lambda g: (g, 0, 0)),
            pl.BlockSpec((1, HID, HID), lambda g: (g, 0, 0)),
            wvec, wvec,
            pl.BlockSpec((1, 3 * HID, HID), lambda g: (g, 0, 0)),
            pl.BlockSpec((1, 3 * HID, HID), lambda g: (g, 0, 0)),
            pl.BlockSpec((1, 1, 3 * HID), lambda g: (g, 0, 0)),
            pl.BlockSpec((1, 1, 3 * HID), lambda g: (g, 0, 0)),
            pl.BlockSpec((1, 64, HID), lambda g: (g, 0, 0)),
            pl.BlockSpec((1, 1, 64), lambda g: (g, 0, 0)),
        ],
        out_specs=pl.BlockSpec((1, NG, 64), lambda g: (g, 0, 0)),
        out_shape=jax.ShapeDtypeStruct((G, NG, 64), F32),
        compiler_params=pltpu.CompilerParams(
            vmem_limit_bytes=100 * 1024 * 1024),
    )
    return f(out0, xs_mol, asn16, batch, molW, attd, molb, Wih, Whh, bih,
             bhh, l2W, l2b)


def _t10(o, fc1W, fc1b, fc2W, fc2b):
    def body(o_r, w1_r, b1_r, w2_r, b2_r, out_r):
        z = jnp.concatenate([o_r[0], o_r[1], o_r[2]], axis=1)
        z1 = jax.nn.relu(_dot(z, w1_r[...].T) + b1_r[...])
        out_r[...] = (jnp.sum(z1 * w2_r[...], axis=1, keepdims=True)
                      + b2_r[...])

    f = pl.pallas_call(
        body,
        grid=(1,),
        in_specs=[
            pl.BlockSpec((G, NG, 64), lambda i: (0, 0, 0)),
            pl.BlockSpec((32, 192), lambda i: (0, 0)),
            pl.BlockSpec((1, 32), lambda i: (0, 0)),
            pl.BlockSpec((1, 32), lambda i: (0, 0)),
            pl.BlockSpec((1, 1), lambda i: (0, 0)),
        ],
        out_specs=pl.BlockSpec((NG, 1), lambda i: (0, 0)),
        out_shape=jax.ShapeDtypeStruct((NG, 1), F32),
    )
    return f(o, fc1W, fc1b, fc2W, fc2b)


def _edge_layer(ex16, table, dsts, srcs):
    denp = _sc_scatter_add(ex16, dsts, NT, 16)
    Gy = _sc_gather(table, srcs, HID)
    slices = _t4(Gy, ex16)
    return [_sc_scatter_add(q, dsts, NT, 32) for q in slices], denp


def kernel(x_inter, edge_index_inter, edge_attr_inter, batch_inter,
           x_intra1, edge_index_intra1, edge_attr_intra1, batch_intra1,
           x_intra2, edge_index_intra2, edge_attr_intra2, batch_intra2,
           params):
    ps = [params['g1'], params['g2'], params['g3']]

    x = jnp.stack([x_inter, x_intra1, x_intra2])
    ea = jnp.stack([edge_attr_inter, edge_attr_intra1, edge_attr_intra2])
    batch = jnp.stack([batch_inter, batch_intra1, batch_intra2])[:, None, :]
    eis = [edge_index_inter, edge_index_intra1, edge_index_intra2]
    srcs = jnp.concatenate([eis[g][0] + g * N for g in range(G)])
    dsts = jnp.concatenate([eis[g][1] + g * N for g in range(G)])

    def st(path, vec=False):
        def get(p):
            v = p
            for k in path:
                v = v[k]
            return v
        s = jnp.stack([get(p) for p in ps])
        return s[:, None, :] if vec else s

    W1 = st(['lin1_W'])
    b1 = st(['lin1_b'], vec=True)
    gateW = st(['gate', 'lin1_W'])
    Wx = gateW[:, :, :HID]
    We = gateW[:, :, HID:]
    attl = st(['gate', 'att_l'], vec=True)
    attr = st(['gate', 'att_r'], vec=True)
    W2g = st(['gate', 'lin2_W'])
    gate_b = st(['gate', 'bias'], vec=True)

    def conv_params(i):
        if i < 2:
            c = [p['atom_convs'][i] for p in ps]
        else:
            c = [p['mol_conv'] for p in ps]
        return (jnp.stack([q['lin_W'] for q in c]),
                jnp.stack([q['att_src'] for q in c])[:, None, :],
                jnp.stack([q['att_dst'] for q in c])[:, None, :],
                jnp.stack([q['bias'] for q in c])[:, None, :])

    def gru_params(i):
        if i == 0:
            g = [p['gru0'] for p in ps]
        elif i < 3:
            g = [p['atom_grus'][i - 1] for p in ps]
        else:
            g = [p['mol_gru'] for p in ps]
        return (jnp.stack([q['W_ih'] for q in g]),
                jnp.stack([q['W_hh'] for q in g]),
                jnp.stack([q['b_ih'] for q in g])[:, None, :],
                jnp.stack([q['b_hh'] for q in g])[:, None, :])

    x1, xg, y2, sd16 = _t1(x, W1, b1, Wx, attr, W2g)
    ew = _t1b(ea, We).reshape(BE, HID)

    Gx = _sc_gather(xg.reshape(NT, HID), srcs, HID)
    sd_e = _sc_gather(sd16.reshape(NT, 16), dsts, 16)
    ex16 = _t2(Gx, ew, sd_e, attl)
    hq, denp = _edge_layer(ex16, y2.reshape(NT, HID), dsts, srcs)

    cW0, cs0, cd0, _ = conv_params(0)
    xcur, xs, as16, ad16 = _t5(hq, denp, gate_b, x1, *gru_params(0),
                               cW0, cs0, cd0)

    for layer in range(2):
        gs = _sc_gather(as16.reshape(NT, 16), srcs, 16)
        gd = _sc_gather(ad16.reshape(NT, 16), dsts, 16)
        ex16 = _t6(gs, gd)
        hq, denp = _edge_layer(ex16, xs.reshape(NT, HID), dsts, srcs)
        cWn, csn, cdn, _ = conv_params(layer + 1)
        _, _, _, conv_bias = conv_params(layer)
        xcur, xs, as16, ad16 = _t5(hq, denp, conv_bias, xcur,
                                   *gru_params(layer + 1), cWn, csn, cdn)

    molW, _, mol_ad, mol_b = conv_params(2)
    out0 = _t8b(xcur, batch)
    o = _t9(out0, xs, as16, batch, molW, mol_ad, mol_b, *gru_params(3),
            st(['lin2_W']), st(['lin2_b'], vec=True))

    return _t10(o, params['fc1_W'], params['fc1_b'].reshape(1, 32),
                params['fc2_W'], params['fc2_b'].reshape(1, 1))
